# baseline (speedup 1.0000x reference)
import functools

import jax
import jax.numpy as jnp
from jax import lax
from jax.experimental import pallas as pl
from jax.experimental.pallas import tpu as pltpu

_MASK_VALUE = -((2.0 - 2.0 ** (-23)) * 2.0 ** 127)
_TINY = 1.1754943508222875e-38

_COLS = 8192


def _threefry2x32(k1, k2, x0, x1):
    rot0 = (13, 15, 26, 6)
    rot1 = (17, 29, 16, 24)
    ks0 = jnp.uint32(k1)
    ks1 = jnp.uint32(k2)
    ks2 = ks0 ^ ks1 ^ jnp.uint32(0x1BD11BDA)

    def rounds(x0, x1, rots):
        for r in rots:
            x0 = x0 + x1
            x1 = (x1 << jnp.uint32(r)) | (x1 >> jnp.uint32(32 - r))
            x1 = x0 ^ x1
        return x0, x1

    x0 = x0 + ks0
    x1 = x1 + ks1
    x0, x1 = rounds(x0, x1, rot0)
    x0 = x0 + ks1
    x1 = x1 + ks2 + jnp.uint32(1)
    x0, x1 = rounds(x0, x1, rot1)
    x0 = x0 + ks2
    x1 = x1 + ks0 + jnp.uint32(2)
    x0, x1 = rounds(x0, x1, rot0)
    x0 = x0 + ks0
    x1 = x1 + ks1 + jnp.uint32(3)
    x0, x1 = rounds(x0, x1, rot1)
    x0 = x0 + ks1
    x1 = x1 + ks2 + jnp.uint32(4)
    x0, x1 = rounds(x0, x1, rot0)
    x0 = x0 + ks2
    x1 = x1 + ks0 + jnp.uint32(5)
    return x0, x1


def _gumbel_block(B, V, C, j):
    r_iota = lax.broadcasted_iota(jnp.int32, (B, C), 0)
    c_iota = lax.broadcasted_iota(jnp.int32, (B, C), 1)
    n = (r_iota * V + j * C + c_iota).astype(jnp.uint32)
    b0, b1 = _threefry2x32(0, 42, jnp.zeros((B, C), jnp.uint32), n)
    bits = b0 ^ b1
    f = lax.bitcast_convert_type(
        (bits >> jnp.uint32(9)) | jnp.uint32(0x3F800000), jnp.float32
    ) - jnp.float32(1.0)
    tiny = jnp.float32(_TINY)
    u = jnp.maximum(tiny, f * (jnp.float32(1.0) - tiny) + tiny)
    return -jnp.log(-jnp.log(u))


def _pass1_body(V, C, logits_ref, mask_ref, m_ref, s_ref):
    j = pl.program_id(0)

    @pl.when(j == 0)
    def _init():
        m_ref[...] = jnp.full_like(m_ref, -jnp.inf)
        s_ref[...] = jnp.zeros_like(s_ref)

    x = logits_ref[...]
    col = j * C + lax.broadcasted_iota(jnp.int32, x.shape, 1)
    valid = jnp.logical_and(mask_ref[...], col < V)
    x1 = jnp.where(valid, x, jnp.float32(_MASK_VALUE))
    bm = jnp.max(x1, axis=1, keepdims=True)
    m_old = m_ref[...]
    m_new = jnp.maximum(m_old, bm)
    bs = jnp.sum(jnp.exp(x1 - m_new), axis=1, keepdims=True)
    s_ref[...] = s_ref[...] * jnp.exp(m_old - m_new) + bs
    m_ref[...] = m_new


def _pass2_body(V, C, logits_ref, mask_ref, m_ref, s_ref,
                probs_ref, act_ref, runmax_ref):
    j = pl.program_id(0)
    B = logits_ref.shape[0]
    lse = m_ref[...] + jnp.log(s_ref[...])

    x = logits_ref[...]
    col = j * C + lax.broadcasted_iota(jnp.int32, (B, C), 1)
    valid = jnp.logical_and(mask_ref[...], col < V)
    norm = jnp.where(valid, x, jnp.float32(_MASK_VALUE)) - lse
    probs_ref[...] = jnp.exp(norm)

    g = _gumbel_block(B, V, C, j)
    val = jnp.where(valid, norm + g, jnp.float32(_MASK_VALUE))
    bm = jnp.max(val, axis=1, keepdims=True)
    cand = jnp.where(val == bm, col, jnp.int32(2**31 - 1))
    bi = jnp.min(cand, axis=1, keepdims=True)

    @pl.when(j == 0)
    def _init():
        runmax_ref[...] = jnp.full_like(runmax_ref, -jnp.inf)
        act_ref[...] = jnp.zeros_like(act_ref)

    better = bm > runmax_ref[...]
    act_ref[...] = jnp.where(better, bi, act_ref[...])
    runmax_ref[...] = jnp.maximum(runmax_ref[...], bm)


def kernel(logits, mask):
    B, V = logits.shape
    C = _COLS if V >= _COLS else V
    nb = (V + C - 1) // C

    row_spec = pl.BlockSpec((B, 1), lambda j: (0, 0))
    blk_spec = pl.BlockSpec((B, C), lambda j: (0, j))

    m, s = pl.pallas_call(
        functools.partial(_pass1_body, V, C),
        grid=(nb,),
        in_specs=[blk_spec, blk_spec],
        out_specs=[row_spec, row_spec],
        out_shape=[
            jax.ShapeDtypeStruct((B, 1), jnp.float32),
            jax.ShapeDtypeStruct((B, 1), jnp.float32),
        ],
    )(logits, mask)

    probs, act = pl.pallas_call(
        functools.partial(_pass2_body, V, C),
        grid=(nb,),
        in_specs=[blk_spec, blk_spec, row_spec, row_spec],
        out_specs=[blk_spec, row_spec],
        out_shape=[
            jax.ShapeDtypeStruct((B, V), jnp.float32),
            jax.ShapeDtypeStruct((B, 1), jnp.int32),
        ],
        scratch_shapes=[pltpu.VMEM((B, 1), jnp.float32)],
    )(logits, mask, m, s)

    return probs, act.reshape(B)
